# baseline (speedup 1.0000x reference)
import functools

import jax
import jax.numpy as jnp
from jax import lax
from jax.experimental import pallas as pl
from jax.experimental.pallas import tpu as pltpu
from jax.experimental.pallas import tpu_sc as plsc

NF = 128
TD = 100
HEADS = 2
QD = NF + TD
HD = QD // HEADS
K = 20
K4 = 24
B = 512

_NC, _NS = 2, 16
_NW = _NC * _NS
_C = 64


_SA, _CA = 8, 32
_SE, _CE = 9, 64


def _ring(idx_v, rows_v, gsems, osems, wid, S, C):
    def g_desc(tab, off, sz, slot):
        return pltpu.make_async_copy(tab.at[idx_v.at[pl.ds(off, sz)]],
                                     rows_v.at[slot, pl.ds(0, sz)],
                                     gsems[slot])

    def o_desc(oh, base, off, sz, slot):
        return pltpu.make_async_copy(rows_v.at[slot, pl.ds(0, sz)],
                                     oh.at[pl.ds(base + off, sz)],
                                     osems[slot])

    def piece_small(tab, ixh, oh, rpw):
        base = wid * rpw
        pltpu.sync_copy(ixh.at[pl.ds(base, rpw)], idx_v.at[pl.ds(0, rpw)])
        chunks = []
        off = 0
        while off < rpw:
            chunks.append((off, min(C, rpw - off)))
            off += C
        nch = len(chunks)
        for ci, (o, sz) in enumerate(chunks[:S]):
            g_desc(tab, o, sz, ci).start()
        nph = (nch + S - 1) // S
        for p in range(nph):
            for b in range(S):
                ci = p * S + b
                if ci >= nch:
                    break
                o, sz = chunks[ci]
                g_desc(tab, o, sz, b).wait()
                o_desc(oh, base, o, sz, b).start()
            for b in range(S):
                nx = (p + 1) * S + b
                if nx >= nch:
                    break
                po, psz = chunks[p * S + b]
                o_desc(oh, base, po, psz, b).wait()
                o2, sz2 = chunks[nx]
                g_desc(tab, o2, sz2, b).start()
        for ci in range(max(0, nch - S), nch):
            o, sz = chunks[ci]
            o_desc(oh, base, o, sz, ci % S).wait()

    def piece_big(tab, ixh, oh, rpw):
        base = wid * rpw
        nch = rpw // C
        nph = nch // S
        pltpu.sync_copy(ixh.at[pl.ds(base, rpw)], idx_v.at[pl.ds(0, rpw)])
        for b in range(S):
            g_desc(tab, b * C, C, b).start()

        def body(p, carry):
            g0 = p * S
            for b in range(S):
                goff = pl.multiple_of((g0 + b) * C, C)
                g_desc(tab, goff, C, b).wait()
                o_desc(oh, base, goff, C, b).start()
            for b in range(S):
                @pl.when(p + 1 < nph)
                def _(b=b, g0=g0):
                    goff = pl.multiple_of((g0 + b) * C, C)
                    o_desc(oh, base, goff, C, b).wait()
                    goff2 = pl.multiple_of((g0 + b + S) * C, C)
                    g_desc(tab, goff2, C, b).start()
            return carry

        lax.fori_loop(0, nph, body, 0)
        for b in range(S):
            goff = (nph - 1) * S * C + b * C
            o_desc(oh, base, goff, C, b).wait()

    return piece_small, piece_big


def _sc_gather_nodes(node_table, edge_table, ixb, ixnb, ixrn, ixnn, ixeb):
    idx_lists = (ixb, ixnb, ixrn, ixnn, ixeb)
    out_type = tuple(jax.ShapeDtypeStruct((ix.shape[0], NF), jnp.float32)
                     for ix in idx_lists)
    rpws = tuple(ix.shape[0] // _NW for ix in idx_lists)
    max_rpw = max(rpws)
    NV = node_table.shape[0]
    nv_full = NV // _CA
    nv_tail = NV - nv_full * _CA
    mesh = plsc.VectorSubcoreMesh(core_axis_name="c", subcore_axis_name="s")

    @functools.partial(
        pl.kernel, mesh=mesh, out_type=out_type,
        scratch_types=[
            pltpu.VMEM((max_rpw,), jnp.int32),
            pltpu.VMEM((_SA, _CA, NF), jnp.float32),
            pltpu.VMEM_SHARED((NV, NF), jnp.float32),
            [pltpu.SemaphoreType.DMA] * _SA,
            [pltpu.SemaphoreType.DMA] * _SA,
        ])
    def run(node_t, edge_t, ixb_h, ixnb_h, ixrn_h, ixnn_h, ixeb_h,
            ob, onb, orn, onn, oeb, idx_v, rows_v, spm, gsems, osems):
        cid = lax.axis_index("c")
        sid = lax.axis_index("s")
        wid = sid * _NC + cid

        def stage_body(i, carry):
            off = pl.multiple_of((sid + i * _NS) * _CA, _CA)
            pltpu.sync_copy(node_t.at[pl.ds(off, _CA)], rows_v.at[0])
            pltpu.sync_copy(rows_v.at[0], spm.at[pl.ds(off, _CA)])
            return carry

        lax.fori_loop(0, (nv_full - sid + _NS - 1) // _NS, stage_body, 0)
        if nv_tail:
            @pl.when(sid == _NS - 1)
            def _():
                off = nv_full * _CA
                pltpu.sync_copy(node_t.at[pl.ds(off, nv_tail)],
                                rows_v.at[0, pl.ds(0, nv_tail)])
                pltpu.sync_copy(rows_v.at[0, pl.ds(0, nv_tail)],
                                spm.at[pl.ds(off, nv_tail)])
        plsc.subcore_barrier()

        piece_small, piece_big = _ring(idx_v, rows_v, gsems, osems, wid,
                                       _SA, _CA)
        piece_small(edge_t, ixeb_h, oeb, rpws[4])
        piece_big(spm, ixnn_h, onn, rpws[3])
        piece_small(spm, ixnb_h, onb, rpws[1])
        piece_small(spm, ixrn_h, orn, rpws[2])
        piece_small(spm, ixb_h, ob, rpws[0])

    return run(node_table, edge_table, ixb, ixnb, ixrn, ixnn, ixeb)


def _sc_gather_edge(edge_table, ixen_part):
    rpw = ixen_part.shape[0] // _NW
    W = edge_table.shape[1]
    dt = edge_table.dtype
    mesh = plsc.VectorSubcoreMesh(core_axis_name="c", subcore_axis_name="s")

    @functools.partial(
        pl.kernel, mesh=mesh,
        out_type=jax.ShapeDtypeStruct((ixen_part.shape[0], W), dt),
        scratch_types=[
            pltpu.VMEM((rpw,), jnp.int32),
            pltpu.VMEM((_SE, _CE, W), dt),
            [pltpu.SemaphoreType.DMA] * _SE,
            [pltpu.SemaphoreType.DMA] * _SE,
        ])
    def run(edge_t, ixen_h, oen, idx_v, rows_v, gsems, osems):
        wid = lax.axis_index("s") * _NC + lax.axis_index("c")
        piece_small, piece_big = _ring(idx_v, rows_v, gsems, osems, wid,
                                       _SE, _CE)
        if rpw % (_SE * _CE) == 0:
            piece_big(edge_t, ixen_h, oen, rpw)
        else:
            piece_small(edge_t, ixen_h, oen, rpw)

    return run(edge_table, ixen_part)


def _conv_body(cf_ref, raw_ref, dt_ref, nf_ref, ef_ref, ids_ref,
               wqn_ref, wqt_ref, wkn_ref, wke_ref, wkt_ref,
               wvn_ref, wve_ref, wvt_ref, wr_ref, br_ref, lng_ref, lnb_ref,
               m1a_ref, m1b2_ref, m1bias_ref, m2w_ref, m2bias_ref,
               tw_ref, tb_ref, out_ref):
    nb = cf_ref.shape[0]
    f32 = jnp.float32
    cf = cf_ref[...]
    tb = tb_ref[...]
    ntf = jnp.cos(tb)
    q = (jnp.dot(cf, wqn_ref[...], preferred_element_type=f32)
         + jnp.dot(ntf, wqt_ref[...], preferred_element_type=f32))
    res = jnp.concatenate([cf, jnp.broadcast_to(ntf, (nb, TD))], axis=1)

    dt = dt_ref[...]
    ttf = jnp.cos(dt * tw_ref[...] + tb)
    nf = nf_ref[...]
    ef = ef_ref[...].astype(f32)
    kf = (jnp.dot(nf, wkn_ref[...], preferred_element_type=f32)
          + jnp.dot(ef, wke_ref[...], preferred_element_type=f32)
          + jnp.dot(ttf, wkt_ref[...], preferred_element_type=f32))
    vf = (jnp.dot(nf, wvn_ref[...], preferred_element_type=f32)
          + jnp.dot(ef, wve_ref[...], preferred_element_type=f32)
          + jnp.dot(ttf, wvt_ref[...], preferred_element_type=f32))
    k3 = kf.reshape(nb, K4, QD)
    v3 = vf.reshape(nb, K4, QD)

    p = q.reshape(nb, 1, QD) * k3
    lane = lax.broadcasted_iota(jnp.int32, (nb, K4, QD), 2)
    h0m = lane < HD
    scale = HD ** -0.5
    a0 = jnp.sum(jnp.where(h0m, p, 0.0), axis=2, keepdims=True) * scale
    a1 = jnp.sum(jnp.where(h0m, 0.0, p), axis=2, keepdims=True) * scale

    ids3 = ids_ref[...].reshape(nb, K4, 1)
    jpos = lax.broadcasted_iota(jnp.int32, (nb, K4, 1), 1)
    ninf = jnp.float32(-jnp.inf)

    def msk(a):
        a = jnp.where(ids3 == 0, -1e10, a)
        return jnp.where(jpos >= K, ninf, a)

    def smax(a):
        m = jnp.max(a, axis=1, keepdims=True)
        e = jnp.exp(a - m)
        return e / jnp.sum(e, axis=1, keepdims=True)

    s0 = smax(msk(a0))
    s1 = smax(msk(a1))
    w3 = jnp.where(h0m, s0, s1)
    v3 = jnp.where(jpos >= K, 0.0, v3)
    o = jnp.sum(w3 * v3, axis=1)

    o = jnp.dot(o, wr_ref[...], preferred_element_type=f32) + br_ref[...] + res
    mu = jnp.mean(o, axis=1, keepdims=True)
    var = jnp.mean((o - mu) ** 2, axis=1, keepdims=True)
    o = lng_ref[...] * (o - mu) / jnp.sqrt(var + 1e-5) + lnb_ref[...]

    h = jnp.maximum(
        jnp.dot(o, m1a_ref[...], preferred_element_type=f32)
        + jnp.dot(raw_ref[...], m1b2_ref[...], preferred_element_type=f32)
        + m1bias_ref[...], 0.0)
    out_ref[...] = jnp.dot(h, m2w_ref[...], preferred_element_type=f32) + m2bias_ref[...]


def _conv(cf, raw, dtf, nf, ef, idsf, tw, tb, Wq, Wk, Wv, Wr, br, lng, lnb,
          m1W, m1b, m2W, m2b, nb=128, n=None, offs=(0, 0, 0, 0, 0, 0),
          interpret=False):
    if n is None:
        n = cf.shape[0]
    assert n % nb == 0
    grid = (n // nb,)
    r2 = lambda a: a.reshape(1, -1)
    wqn, wqt = Wq[:NF], Wq[NF:]
    wkn, wke, wkt = Wk[:NF], Wk[NF:2 * NF], Wk[2 * NF:]
    wvn, wve, wvt = Wv[:NF], Wv[NF:2 * NF], Wv[2 * NF:]
    m1a, m1b2 = m1W[:QD], m1W[QD:]

    def bs_c(o):
        return pl.BlockSpec((nb, NF), lambda i, o=o: (i + o, 0))

    def bs_f(o):
        return pl.BlockSpec((nb * K4, NF), lambda i, o=o: (i + o, 0))

    def bs_d(o):
        return pl.BlockSpec((nb * K4, 1), lambda i, o=o: (i + o, 0))

    def bw(a):
        shape = a.shape
        return pl.BlockSpec(shape, lambda i: (0,) * len(shape))

    weights = (wqn, wqt, wkn, wke, wkt, wvn, wve, wvt, Wr, r2(br), r2(lng),
               r2(lnb), m1a, m1b2, r2(m1b), m2W, r2(m2b), r2(tw), r2(tb))
    specs = [bs_c(offs[0]), bs_c(offs[1]), bs_d(offs[2]), bs_f(offs[3]),
             bs_f(offs[4]), bs_d(offs[5])]
    return pl.pallas_call(
        _conv_body,
        grid=grid,
        in_specs=specs + [bw(w) for w in weights],
        out_specs=pl.BlockSpec((nb, NF), lambda i: (i, 0)),
        out_shape=jax.ShapeDtypeStruct((n, NF), jnp.float32),
        interpret=interpret,
    )(cf, raw, dtf, nf, ef, idsf, *weights)


def kernel(node_ids, node_interact_times, nbr_b_ids, nbr_b_eids, nbr_b_times,
           nbr_n_ids, nbr_n_eids, nbr_n_times, node_table, edge_table,
           time_w, time_b,
           Wq0, Wk0, Wv0, Wr0, br0, lng0, lnb0, m1W0, m1b0, m2W0, m2b0,
           Wq1, Wk1, Wv1, Wr1, br1, lng1, lnb1, m1W1, m1b1, m2W1, m2b1):
    i32 = jnp.int32
    pad = lambda a: jnp.pad(a, ((0, 0), (0, K4 - K)))
    ixb = node_ids.astype(i32)
    nb_ids = nbr_b_ids.astype(i32)
    ixnb = pad(nb_ids).reshape(-1)
    ixrn = nb_ids.reshape(-1)
    ixnn = pad(nbr_n_ids.astype(i32)).reshape(-1)
    ixeb = pad(nbr_b_eids.astype(i32)).reshape(-1)
    ixen = pad(nbr_n_eids.astype(i32)).reshape(-1)

    ob, onb, orn, onn, oeb = _sc_gather_nodes(
        node_table, edge_table, ixb, ixnb, ixrn, ixnn, ixeb)

    parts_c = (3840, 3840, 1280, 1280)
    starts_c = (0, 3840, 7680, 8960)
    oen_parts = [
        _sc_gather_edge(edge_table, ixen[s * K4:(s + c) * K4])
        for s, c in zip(starts_c, parts_c)]

    dtb = (node_interact_times[:, None] - pad(nbr_b_times)).reshape(-1, 1)
    dtn = (nbr_b_times.reshape(-1)[:, None] - pad(nbr_n_times)).reshape(-1, 1)
    idsb_f = ixnb.reshape(-1, 1)
    idsn_f = ixnn.reshape(-1, 1)

    p0 = (Wq0, Wk0, Wv0, Wr0, br0, lng0, lnb0, m1W0, m1b0, m2W0, m2b0)
    p1 = (Wq1, Wk1, Wv1, Wr1, br1, lng1, lnb1, m1W1, m1b1, m2W1, m2b1)

    conv_b = _conv(ob, ob, dtb, onb, oeb, idsb_f, time_w, time_b, *p0)
    nbp = 128
    outs = []
    for p, (s, c) in enumerate(zip(starts_c, parts_c)):
        bo = s // nbp
        cn_p = _conv(orn, orn, dtn, onn, oen_parts[p], idsn_f, time_w, time_b,
                     *p0, nb=nbp, n=c, offs=(bo, bo, bo, bo, 0, bo))
        bq = c // K
        b0 = (s // K) // bq
        cn3_p = jnp.pad(cn_p.reshape(bq, K, NF),
                        ((0, 0), (0, K4 - K), (0, 0))).reshape(-1, NF)
        outs.append(_conv(conv_b, ob, dtb, cn3_p, oeb, idsb_f, time_w, time_b,
                          *p1, nb=bq, n=bq,
                          offs=(b0, b0, b0, 0, b0, b0)))
    return jnp.concatenate(outs, axis=0)


# speedup vs baseline: 1.0479x; 1.0479x over previous
import functools

import jax
import jax.numpy as jnp
from jax import lax
from jax.experimental import pallas as pl
from jax.experimental.pallas import tpu as pltpu
from jax.experimental.pallas import tpu_sc as plsc

NF = 128
TD = 100
HEADS = 2
QD = NF + TD
HD = QD // HEADS
K = 20
K4 = 24
B = 512

_NC, _NS = 2, 16
_NW = _NC * _NS
_C = 64


_SA, _CA = 8, 32
_SE, _CE = 10, 64


def _ring(idx_v, rows_v, gsems, osems, wid, S, C):
    def g_desc(tab, off, sz, slot):
        return pltpu.make_async_copy(tab.at[idx_v.at[pl.ds(off, sz)]],
                                     rows_v.at[slot, pl.ds(0, sz)],
                                     gsems[slot])

    def o_desc(oh, base, off, sz, slot):
        return pltpu.make_async_copy(rows_v.at[slot, pl.ds(0, sz)],
                                     oh.at[pl.ds(base + off, sz)],
                                     osems[slot])

    def piece_small(tab, ixh, oh, rpw):
        base = wid * rpw
        pltpu.sync_copy(ixh.at[pl.ds(base, rpw)], idx_v.at[pl.ds(0, rpw)])
        chunks = []
        off = 0
        while off < rpw:
            chunks.append((off, min(C, rpw - off)))
            off += C
        nch = len(chunks)
        for ci, (o, sz) in enumerate(chunks[:S]):
            g_desc(tab, o, sz, ci).start()
        nph = (nch + S - 1) // S
        for p in range(nph):
            for b in range(S):
                ci = p * S + b
                if ci >= nch:
                    break
                o, sz = chunks[ci]
                g_desc(tab, o, sz, b).wait()
                o_desc(oh, base, o, sz, b).start()
            for b in range(S):
                nx = (p + 1) * S + b
                if nx >= nch:
                    break
                po, psz = chunks[p * S + b]
                o_desc(oh, base, po, psz, b).wait()
                o2, sz2 = chunks[nx]
                g_desc(tab, o2, sz2, b).start()
        for ci in range(max(0, nch - S), nch):
            o, sz = chunks[ci]
            o_desc(oh, base, o, sz, ci % S).wait()

    def piece_big(tab, ixh, oh, rpw):
        base = wid * rpw
        nch = rpw // C
        nph = nch // S
        pltpu.sync_copy(ixh.at[pl.ds(base, rpw)], idx_v.at[pl.ds(0, rpw)])
        for b in range(S):
            g_desc(tab, b * C, C, b).start()

        def body(p, carry):
            g0 = p * S
            for b in range(S):
                goff = pl.multiple_of((g0 + b) * C, C)
                g_desc(tab, goff, C, b).wait()
                o_desc(oh, base, goff, C, b).start()
            for b in range(S):
                @pl.when(p + 1 < nph)
                def _(b=b, g0=g0):
                    goff = pl.multiple_of((g0 + b) * C, C)
                    o_desc(oh, base, goff, C, b).wait()
                    goff2 = pl.multiple_of((g0 + b + S) * C, C)
                    g_desc(tab, goff2, C, b).start()
            return carry

        lax.fori_loop(0, nph, body, 0)
        for b in range(S):
            goff = (nph - 1) * S * C + b * C
            o_desc(oh, base, goff, C, b).wait()

    return piece_small, piece_big


def _sc_gather_nodes(node_table, edge_table, ixb, ixnb, ixrn, ixnn, ixeb):
    idx_lists = (ixb, ixnb, ixrn, ixnn, ixeb)
    out_type = tuple(jax.ShapeDtypeStruct((ix.shape[0], NF), jnp.float32)
                     for ix in idx_lists)
    rpws = tuple(ix.shape[0] // _NW for ix in idx_lists)
    max_rpw = max(rpws)
    NV = node_table.shape[0]
    nv_full = NV // _CA
    nv_tail = NV - nv_full * _CA
    mesh = plsc.VectorSubcoreMesh(core_axis_name="c", subcore_axis_name="s")

    @functools.partial(
        pl.kernel, mesh=mesh, out_type=out_type,
        scratch_types=[
            pltpu.VMEM((max_rpw,), jnp.int32),
            pltpu.VMEM((_SA, _CA, NF), jnp.float32),
            pltpu.VMEM_SHARED((NV, NF), jnp.float32),
            [pltpu.SemaphoreType.DMA] * _SA,
            [pltpu.SemaphoreType.DMA] * _SA,
        ])
    def run(node_t, edge_t, ixb_h, ixnb_h, ixrn_h, ixnn_h, ixeb_h,
            ob, onb, orn, onn, oeb, idx_v, rows_v, spm, gsems, osems):
        cid = lax.axis_index("c")
        sid = lax.axis_index("s")
        wid = sid * _NC + cid

        def stage_body(i, carry):
            off = pl.multiple_of((sid + i * _NS) * _CA, _CA)
            pltpu.sync_copy(node_t.at[pl.ds(off, _CA)], rows_v.at[0])
            pltpu.sync_copy(rows_v.at[0], spm.at[pl.ds(off, _CA)])
            return carry

        lax.fori_loop(0, (nv_full - sid + _NS - 1) // _NS, stage_body, 0)
        if nv_tail:
            @pl.when(sid == _NS - 1)
            def _():
                off = nv_full * _CA
                pltpu.sync_copy(node_t.at[pl.ds(off, nv_tail)],
                                rows_v.at[0, pl.ds(0, nv_tail)])
                pltpu.sync_copy(rows_v.at[0, pl.ds(0, nv_tail)],
                                spm.at[pl.ds(off, nv_tail)])
        plsc.subcore_barrier()

        piece_small, piece_big = _ring(idx_v, rows_v, gsems, osems, wid,
                                       _SA, _CA)
        piece_small(edge_t, ixeb_h, oeb, rpws[4])
        piece_big(spm, ixnn_h, onn, rpws[3])
        piece_small(spm, ixnb_h, onb, rpws[1])
        piece_small(spm, ixrn_h, orn, rpws[2])
        piece_small(spm, ixb_h, ob, rpws[0])

    return run(node_table, edge_table, ixb, ixnb, ixrn, ixnn, ixeb)


def _sc_gather_edge(edge_table, ixen_part):
    rpw = ixen_part.shape[0] // _NW
    W = edge_table.shape[1]
    dt = edge_table.dtype
    mesh = plsc.VectorSubcoreMesh(core_axis_name="c", subcore_axis_name="s")

    @functools.partial(
        pl.kernel, mesh=mesh,
        out_type=jax.ShapeDtypeStruct((ixen_part.shape[0], W), dt),
        scratch_types=[
            pltpu.VMEM((rpw,), jnp.int32),
            pltpu.VMEM((_SE, _CE, W), dt),
            [pltpu.SemaphoreType.DMA] * _SE,
            [pltpu.SemaphoreType.DMA] * _SE,
        ])
    def run(edge_t, ixen_h, oen, idx_v, rows_v, gsems, osems):
        wid = lax.axis_index("s") * _NC + lax.axis_index("c")
        piece_small, piece_big = _ring(idx_v, rows_v, gsems, osems, wid,
                                       _SE, _CE)
        if rpw % (_SE * _CE) == 0:
            piece_big(edge_t, ixen_h, oen, rpw)
        else:
            piece_small(edge_t, ixen_h, oen, rpw)

    return run(edge_table, ixen_part)


def _conv_body(cf_ref, raw_ref, dt_ref, nf_ref, ef_ref, ids_ref,
               wqn_ref, wqt_ref, wkn_ref, wke_ref, wkt_ref,
               wvn_ref, wve_ref, wvt_ref, wr_ref, br_ref, lng_ref, lnb_ref,
               m1a_ref, m1b2_ref, m1bias_ref, m2w_ref, m2bias_ref,
               tw_ref, tb_ref, out_ref):
    nb = cf_ref.shape[0]
    f32 = jnp.float32
    cf = cf_ref[...]
    tb = tb_ref[...]
    ntf = jnp.cos(tb)
    q = (jnp.dot(cf, wqn_ref[...], preferred_element_type=f32)
         + jnp.dot(ntf, wqt_ref[...], preferred_element_type=f32))
    res = jnp.concatenate([cf, jnp.broadcast_to(ntf, (nb, TD))], axis=1)

    dt = dt_ref[...]
    ttf = jnp.cos(dt * tw_ref[...] + tb)
    nf = nf_ref[...]
    ef = ef_ref[...].astype(f32)
    kf = (jnp.dot(nf, wkn_ref[...], preferred_element_type=f32)
          + jnp.dot(ef, wke_ref[...], preferred_element_type=f32)
          + jnp.dot(ttf, wkt_ref[...], preferred_element_type=f32))
    vf = (jnp.dot(nf, wvn_ref[...], preferred_element_type=f32)
          + jnp.dot(ef, wve_ref[...], preferred_element_type=f32)
          + jnp.dot(ttf, wvt_ref[...], preferred_element_type=f32))
    k3 = kf.reshape(nb, K4, QD)
    v3 = vf.reshape(nb, K4, QD)

    p = q.reshape(nb, 1, QD) * k3
    lane = lax.broadcasted_iota(jnp.int32, (nb, K4, QD), 2)
    h0m = lane < HD
    scale = HD ** -0.5
    a0 = jnp.sum(jnp.where(h0m, p, 0.0), axis=2, keepdims=True) * scale
    a1 = jnp.sum(jnp.where(h0m, 0.0, p), axis=2, keepdims=True) * scale

    ids3 = ids_ref[...].reshape(nb, K4, 1)
    jpos = lax.broadcasted_iota(jnp.int32, (nb, K4, 1), 1)
    ninf = jnp.float32(-jnp.inf)

    def msk(a):
        a = jnp.where(ids3 == 0, -1e10, a)
        return jnp.where(jpos >= K, ninf, a)

    def smax(a):
        m = jnp.max(a, axis=1, keepdims=True)
        e = jnp.exp(a - m)
        return e / jnp.sum(e, axis=1, keepdims=True)

    s0 = smax(msk(a0))
    s1 = smax(msk(a1))
    w3 = jnp.where(h0m, s0, s1)
    v3 = jnp.where(jpos >= K, 0.0, v3)
    o = jnp.sum(w3 * v3, axis=1)

    o = jnp.dot(o, wr_ref[...], preferred_element_type=f32) + br_ref[...] + res
    mu = jnp.mean(o, axis=1, keepdims=True)
    var = jnp.mean((o - mu) ** 2, axis=1, keepdims=True)
    o = lng_ref[...] * (o - mu) / jnp.sqrt(var + 1e-5) + lnb_ref[...]

    h = jnp.maximum(
        jnp.dot(o, m1a_ref[...], preferred_element_type=f32)
        + jnp.dot(raw_ref[...], m1b2_ref[...], preferred_element_type=f32)
        + m1bias_ref[...], 0.0)
    out_ref[...] = jnp.dot(h, m2w_ref[...], preferred_element_type=f32) + m2bias_ref[...]


def _conv(cf, raw, dtf, nf, ef, idsf, tw, tb, Wq, Wk, Wv, Wr, br, lng, lnb,
          m1W, m1b, m2W, m2b, nb=128, n=None, offs=(0, 0, 0, 0, 0, 0),
          interpret=False):
    if n is None:
        n = cf.shape[0]
    assert n % nb == 0
    grid = (n // nb,)
    r2 = lambda a: a.reshape(1, -1)
    wqn, wqt = Wq[:NF], Wq[NF:]
    wkn, wke, wkt = Wk[:NF], Wk[NF:2 * NF], Wk[2 * NF:]
    wvn, wve, wvt = Wv[:NF], Wv[NF:2 * NF], Wv[2 * NF:]
    m1a, m1b2 = m1W[:QD], m1W[QD:]

    def bs_c(o):
        return pl.BlockSpec((nb, NF), lambda i, o=o: (i + o, 0))

    def bs_f(o):
        return pl.BlockSpec((nb * K4, NF), lambda i, o=o: (i + o, 0))

    def bs_d(o):
        return pl.BlockSpec((nb * K4, 1), lambda i, o=o: (i + o, 0))

    def bw(a):
        shape = a.shape
        return pl.BlockSpec(shape, lambda i: (0,) * len(shape))

    weights = (wqn, wqt, wkn, wke, wkt, wvn, wve, wvt, Wr, r2(br), r2(lng),
               r2(lnb), m1a, m1b2, r2(m1b), m2W, r2(m2b), r2(tw), r2(tb))
    specs = [bs_c(offs[0]), bs_c(offs[1]), bs_d(offs[2]), bs_f(offs[3]),
             bs_f(offs[4]), bs_d(offs[5])]
    return pl.pallas_call(
        _conv_body,
        grid=grid,
        in_specs=specs + [bw(w) for w in weights],
        out_specs=pl.BlockSpec((nb, NF), lambda i: (i, 0)),
        out_shape=jax.ShapeDtypeStruct((n, NF), jnp.float32),
        interpret=interpret,
    )(cf, raw, dtf, nf, ef, idsf, *weights)


def kernel(node_ids, node_interact_times, nbr_b_ids, nbr_b_eids, nbr_b_times,
           nbr_n_ids, nbr_n_eids, nbr_n_times, node_table, edge_table,
           time_w, time_b,
           Wq0, Wk0, Wv0, Wr0, br0, lng0, lnb0, m1W0, m1b0, m2W0, m2b0,
           Wq1, Wk1, Wv1, Wr1, br1, lng1, lnb1, m1W1, m1b1, m2W1, m2b1):
    i32 = jnp.int32
    pad = lambda a: jnp.pad(a, ((0, 0), (0, K4 - K)))
    ixb = node_ids.astype(i32)
    nb_ids = nbr_b_ids.astype(i32)
    ixnb = pad(nb_ids).reshape(-1)
    ixrn = nb_ids.reshape(-1)
    ixnn = pad(nbr_n_ids.astype(i32)).reshape(-1)
    ixeb = pad(nbr_b_eids.astype(i32)).reshape(-1)
    ixen = pad(nbr_n_eids.astype(i32)).reshape(-1)

    ob, onb, orn, onn, oeb = _sc_gather_nodes(
        node_table, edge_table, ixb, ixnb, ixrn, ixnn, ixeb)

    parts_c = (2560, 2560, 2560, 2560)
    starts_c = (0, 2560, 5120, 7680)
    oen_parts = [
        _sc_gather_edge(edge_table, ixen[s * K4:(s + c) * K4])
        for s, c in zip(starts_c, parts_c)]

    dtb = (node_interact_times[:, None] - pad(nbr_b_times)).reshape(-1, 1)
    dtn = (nbr_b_times.reshape(-1)[:, None] - pad(nbr_n_times)).reshape(-1, 1)
    idsb_f = ixnb.reshape(-1, 1)
    idsn_f = ixnn.reshape(-1, 1)

    p0 = (Wq0, Wk0, Wv0, Wr0, br0, lng0, lnb0, m1W0, m1b0, m2W0, m2b0)
    p1 = (Wq1, Wk1, Wv1, Wr1, br1, lng1, lnb1, m1W1, m1b1, m2W1, m2b1)

    conv_b = _conv(ob, ob, dtb, onb, oeb, idsb_f, time_w, time_b, *p0)
    nbp = 128
    outs = []
    for p, (s, c) in enumerate(zip(starts_c, parts_c)):
        bo = s // nbp
        cn_p = _conv(orn, orn, dtn, onn, oen_parts[p], idsn_f, time_w, time_b,
                     *p0, nb=nbp, n=c, offs=(bo, bo, bo, bo, 0, bo))
        bq = c // K
        b0 = (s // K) // bq
        cn3_p = jnp.pad(cn_p.reshape(bq, K, NF),
                        ((0, 0), (0, K4 - K), (0, 0))).reshape(-1, NF)
        outs.append(_conv(conv_b, ob, dtb, cn3_p, oeb, idsb_f, time_w, time_b,
                          *p1, nb=bq, n=bq,
                          offs=(b0, b0, b0, 0, b0, b0)))
    return jnp.concatenate(outs, axis=0)


# speedup vs baseline: 1.0483x; 1.0003x over previous
import functools

import jax
import jax.numpy as jnp
from jax import lax
from jax.experimental import pallas as pl
from jax.experimental.pallas import tpu as pltpu
from jax.experimental.pallas import tpu_sc as plsc

NF = 128
TD = 100
HEADS = 2
QD = NF + TD
HD = QD // HEADS
K = 20
K4 = 24
B = 512

_NC, _NS = 2, 16
_NW = _NC * _NS


_SA, _CA = 8, 32
_SE, _CE = 10, 64


def _ring(idx_v, rows_v, gsems, osems, wid, S, C):
    def g_desc(tab, off, sz, slot):
        return pltpu.make_async_copy(tab.at[idx_v.at[pl.ds(off, sz)]],
                                     rows_v.at[slot, pl.ds(0, sz)],
                                     gsems[slot])

    def o_desc(oh, base, off, sz, slot):
        return pltpu.make_async_copy(rows_v.at[slot, pl.ds(0, sz)],
                                     oh.at[pl.ds(base + off, sz)],
                                     osems[slot])

    def piece_small(tab, ixh, oh, rpw):
        base = wid * rpw
        pltpu.sync_copy(ixh.at[pl.ds(base, rpw)], idx_v.at[pl.ds(0, rpw)])
        chunks = []
        off = 0
        while off < rpw:
            chunks.append((off, min(C, rpw - off)))
            off += C
        nch = len(chunks)
        for ci, (o, sz) in enumerate(chunks[:S]):
            g_desc(tab, o, sz, ci).start()
        nph = (nch + S - 1) // S
        for p in range(nph):
            for b in range(S):
                ci = p * S + b
                if ci >= nch:
                    break
                o, sz = chunks[ci]
                g_desc(tab, o, sz, b).wait()
                o_desc(oh, base, o, sz, b).start()
            for b in range(S):
                nx = (p + 1) * S + b
                if nx >= nch:
                    break
                po, psz = chunks[p * S + b]
                o_desc(oh, base, po, psz, b).wait()
                o2, sz2 = chunks[nx]
                g_desc(tab, o2, sz2, b).start()
        for ci in range(max(0, nch - S), nch):
            o, sz = chunks[ci]
            o_desc(oh, base, o, sz, ci % S).wait()

    def piece_big(tab, ixh, oh, rpw):
        base = wid * rpw
        nch = rpw // C
        nph = nch // S
        pltpu.sync_copy(ixh.at[pl.ds(base, rpw)], idx_v.at[pl.ds(0, rpw)])
        for b in range(S):
            g_desc(tab, b * C, C, b).start()

        def body(p, carry):
            g0 = p * S
            for b in range(S):
                goff = pl.multiple_of((g0 + b) * C, C)
                g_desc(tab, goff, C, b).wait()
                o_desc(oh, base, goff, C, b).start()
            for b in range(S):
                @pl.when(p + 1 < nph)
                def _(b=b, g0=g0):
                    goff = pl.multiple_of((g0 + b) * C, C)
                    o_desc(oh, base, goff, C, b).wait()
                    goff2 = pl.multiple_of((g0 + b + S) * C, C)
                    g_desc(tab, goff2, C, b).start()
            return carry

        lax.fori_loop(0, nph, body, 0)
        for b in range(S):
            goff = (nph - 1) * S * C + b * C
            o_desc(oh, base, goff, C, b).wait()

    return piece_small, piece_big


def _sc_gather_nodes(node_table, edge_table, ixb, ixnb, ixrn, ixnn, ixeb):
    idx_lists = (ixb, ixnb, ixrn, ixnn, ixeb)
    out_type = tuple(jax.ShapeDtypeStruct((ix.shape[0], NF), jnp.float32)
                     for ix in idx_lists)
    rpws = tuple(ix.shape[0] // _NW for ix in idx_lists)
    max_rpw = max(rpws)
    NV = node_table.shape[0]
    nv_full = NV // _CA
    nv_tail = NV - nv_full * _CA
    mesh = plsc.VectorSubcoreMesh(core_axis_name="c", subcore_axis_name="s")

    @functools.partial(
        pl.kernel, mesh=mesh, out_type=out_type,
        scratch_types=[
            pltpu.VMEM((max_rpw,), jnp.int32),
            pltpu.VMEM((_SA, _CA, NF), jnp.float32),
            pltpu.VMEM_SHARED((NV, NF), jnp.float32),
            [pltpu.SemaphoreType.DMA] * _SA,
            [pltpu.SemaphoreType.DMA] * _SA,
        ])
    def run(node_t, edge_t, ixb_h, ixnb_h, ixrn_h, ixnn_h, ixeb_h,
            ob, onb, orn, onn, oeb, idx_v, rows_v, spm, gsems, osems):
        cid = lax.axis_index("c")
        sid = lax.axis_index("s")
        wid = sid * _NC + cid

        def stage_body(i, carry):
            off = pl.multiple_of((sid + i * _NS) * _CA, _CA)
            pltpu.sync_copy(node_t.at[pl.ds(off, _CA)], rows_v.at[0])
            pltpu.sync_copy(rows_v.at[0], spm.at[pl.ds(off, _CA)])
            return carry

        lax.fori_loop(0, (nv_full - sid + _NS - 1) // _NS, stage_body, 0)
        if nv_tail:
            @pl.when(sid == _NS - 1)
            def _():
                off = nv_full * _CA
                pltpu.sync_copy(node_t.at[pl.ds(off, nv_tail)],
                                rows_v.at[0, pl.ds(0, nv_tail)])
                pltpu.sync_copy(rows_v.at[0, pl.ds(0, nv_tail)],
                                spm.at[pl.ds(off, nv_tail)])
        plsc.subcore_barrier()

        piece_small, piece_big = _ring(idx_v, rows_v, gsems, osems, wid,
                                       _SA, _CA)
        piece_small(edge_t, ixeb_h, oeb, rpws[4])
        piece_big(spm, ixnn_h, onn, rpws[3])
        piece_small(spm, ixnb_h, onb, rpws[1])
        piece_small(spm, ixrn_h, orn, rpws[2])
        piece_small(spm, ixb_h, ob, rpws[0])

    return run(node_table, edge_table, ixb, ixnb, ixrn, ixnn, ixeb)


def _sc_gather_edge(edge_table, ixen_part):
    rpw = ixen_part.shape[0] // _NW
    W = edge_table.shape[1]
    dt = edge_table.dtype
    mesh = plsc.VectorSubcoreMesh(core_axis_name="c", subcore_axis_name="s")

    @functools.partial(
        pl.kernel, mesh=mesh,
        out_type=jax.ShapeDtypeStruct((ixen_part.shape[0], W), dt),
        scratch_types=[
            pltpu.VMEM((rpw,), jnp.int32),
            pltpu.VMEM((_SE, _CE, W), dt),
            [pltpu.SemaphoreType.DMA] * _SE,
            [pltpu.SemaphoreType.DMA] * _SE,
        ])
    def run(edge_t, ixen_h, oen, idx_v, rows_v, gsems, osems):
        wid = lax.axis_index("s") * _NC + lax.axis_index("c")
        piece_small, piece_big = _ring(idx_v, rows_v, gsems, osems, wid,
                                       _SE, _CE)
        if rpw % (_SE * _CE) == 0:
            piece_big(edge_t, ixen_h, oen, rpw)
        else:
            piece_small(edge_t, ixen_h, oen, rpw)

    return run(edge_table, ixen_part)


def _conv_body(cf_ref, raw_ref, dt_ref, nf_ref, ef_ref, ids_ref,
               wqn_ref, wqt_ref, wkn_ref, wke_ref, wkt_ref,
               wvn_ref, wve_ref, wvt_ref, wr_ref, br_ref, lng_ref, lnb_ref,
               m1a_ref, m1b2_ref, m1bias_ref, m2w_ref, m2bias_ref,
               tw_ref, tb_ref, out_ref):
    nb = cf_ref.shape[0]
    f32 = jnp.float32
    cf = cf_ref[...]
    tb = tb_ref[...]
    ntf = jnp.cos(tb)
    q = (jnp.dot(cf, wqn_ref[...], preferred_element_type=f32)
         + jnp.dot(ntf, wqt_ref[...], preferred_element_type=f32))
    res = jnp.concatenate([cf, jnp.broadcast_to(ntf, (nb, TD))], axis=1)

    dt = dt_ref[...]
    ttf = jnp.cos(dt * tw_ref[...] + tb)
    nf = nf_ref[...]
    ef = ef_ref[...].astype(f32)
    kf = (jnp.dot(nf, wkn_ref[...], preferred_element_type=f32)
          + jnp.dot(ef, wke_ref[...], preferred_element_type=f32)
          + jnp.dot(ttf, wkt_ref[...], preferred_element_type=f32))
    vf = (jnp.dot(nf, wvn_ref[...], preferred_element_type=f32)
          + jnp.dot(ef, wve_ref[...], preferred_element_type=f32)
          + jnp.dot(ttf, wvt_ref[...], preferred_element_type=f32))
    k3 = kf.reshape(nb, K4, QD)
    v3 = vf.reshape(nb, K4, QD)

    p = q.reshape(nb, 1, QD) * k3
    lane = lax.broadcasted_iota(jnp.int32, (nb, K4, QD), 2)
    h0m = lane < HD
    scale = HD ** -0.5
    a0 = jnp.sum(jnp.where(h0m, p, 0.0), axis=2, keepdims=True) * scale
    a1 = jnp.sum(jnp.where(h0m, 0.0, p), axis=2, keepdims=True) * scale

    ids3 = ids_ref[...].reshape(nb, K4, 1)
    jpos = lax.broadcasted_iota(jnp.int32, (nb, K4, 1), 1)
    ninf = jnp.float32(-jnp.inf)

    def msk(a):
        a = jnp.where(ids3 == 0, -1e10, a)
        return jnp.where(jpos >= K, ninf, a)

    def smax(a):
        m = jnp.max(a, axis=1, keepdims=True)
        e = jnp.exp(a - m)
        return e / jnp.sum(e, axis=1, keepdims=True)

    s0 = smax(msk(a0))
    s1 = smax(msk(a1))
    w3 = jnp.where(h0m, s0, s1)
    v3 = jnp.where(jpos >= K, 0.0, v3)
    o = jnp.sum(w3 * v3, axis=1)

    o = jnp.dot(o, wr_ref[...], preferred_element_type=f32) + br_ref[...] + res
    mu = jnp.mean(o, axis=1, keepdims=True)
    var = jnp.mean((o - mu) ** 2, axis=1, keepdims=True)
    o = lng_ref[...] * (o - mu) / jnp.sqrt(var + 1e-5) + lnb_ref[...]

    h = jnp.maximum(
        jnp.dot(o, m1a_ref[...], preferred_element_type=f32)
        + jnp.dot(raw_ref[...], m1b2_ref[...], preferred_element_type=f32)
        + m1bias_ref[...], 0.0)
    out_ref[...] = jnp.dot(h, m2w_ref[...], preferred_element_type=f32) + m2bias_ref[...]


def _conv(cf, raw, dtf, nf, ef, idsf, tw, tb, Wq, Wk, Wv, Wr, br, lng, lnb,
          m1W, m1b, m2W, m2b, nb=128, n=None, offs=(0, 0, 0, 0, 0, 0),
          interpret=False):
    if n is None:
        n = cf.shape[0]
    assert n % nb == 0
    grid = (n // nb,)
    r2 = lambda a: a.reshape(1, -1)
    wqn, wqt = Wq[:NF], Wq[NF:]
    wkn, wke, wkt = Wk[:NF], Wk[NF:2 * NF], Wk[2 * NF:]
    wvn, wve, wvt = Wv[:NF], Wv[NF:2 * NF], Wv[2 * NF:]
    m1a, m1b2 = m1W[:QD], m1W[QD:]

    def bs_c(o):
        return pl.BlockSpec((nb, NF), lambda i, o=o: (i + o, 0))

    def bs_f(o):
        return pl.BlockSpec((nb * K4, NF), lambda i, o=o: (i + o, 0))

    def bs_d(o):
        return pl.BlockSpec((nb * K4, 1), lambda i, o=o: (i + o, 0))

    def bw(a):
        shape = a.shape
        return pl.BlockSpec(shape, lambda i: (0,) * len(shape))

    weights = (wqn, wqt, wkn, wke, wkt, wvn, wve, wvt, Wr, r2(br), r2(lng),
               r2(lnb), m1a, m1b2, r2(m1b), m2W, r2(m2b), r2(tw), r2(tb))
    specs = [bs_c(offs[0]), bs_c(offs[1]), bs_d(offs[2]), bs_f(offs[3]),
             bs_f(offs[4]), bs_d(offs[5])]
    return pl.pallas_call(
        _conv_body,
        grid=grid,
        in_specs=specs + [bw(w) for w in weights],
        out_specs=pl.BlockSpec((nb, NF), lambda i: (i, 0)),
        out_shape=jax.ShapeDtypeStruct((n, NF), jnp.float32),
        interpret=interpret,
    )(cf, raw, dtf, nf, ef, idsf, *weights)


def kernel(node_ids, node_interact_times, nbr_b_ids, nbr_b_eids, nbr_b_times,
           nbr_n_ids, nbr_n_eids, nbr_n_times, node_table, edge_table,
           time_w, time_b,
           Wq0, Wk0, Wv0, Wr0, br0, lng0, lnb0, m1W0, m1b0, m2W0, m2b0,
           Wq1, Wk1, Wv1, Wr1, br1, lng1, lnb1, m1W1, m1b1, m2W1, m2b1):
    i32 = jnp.int32
    pad = lambda a: jnp.pad(a, ((0, 0), (0, K4 - K)))
    ixb = node_ids.astype(i32)
    nb_ids = nbr_b_ids.astype(i32)
    ixnb = pad(nb_ids).reshape(-1)
    ixrn = nb_ids.reshape(-1)
    ixnn = pad(nbr_n_ids.astype(i32)).reshape(-1)
    ixeb = pad(nbr_b_eids.astype(i32)).reshape(-1)
    ixen = pad(nbr_n_eids.astype(i32)).reshape(-1)

    ob, onb, orn, onn, oeb = _sc_gather_nodes(
        node_table, edge_table, ixb, ixnb, ixrn, ixnn, ixeb)

    parts_c = (2560, 2560, 2560, 2560)
    starts_c = (0, 2560, 5120, 7680)
    oen_parts = [
        _sc_gather_edge(edge_table, ixen[s * K4:(s + c) * K4])
        for s, c in zip(starts_c, parts_c)]

    dtb = (node_interact_times[:, None] - pad(nbr_b_times)).reshape(-1, 1)
    dtn = (nbr_b_times.reshape(-1)[:, None] - pad(nbr_n_times)).reshape(-1, 1)
    idsb_f = ixnb.reshape(-1, 1)
    idsn_f = ixnn.reshape(-1, 1)

    p0 = (Wq0, Wk0, Wv0, Wr0, br0, lng0, lnb0, m1W0, m1b0, m2W0, m2b0)
    p1 = (Wq1, Wk1, Wv1, Wr1, br1, lng1, lnb1, m1W1, m1b1, m2W1, m2b1)

    conv_b = _conv(ob, ob, dtb, onb, oeb, idsb_f, time_w, time_b, *p0)
    nbp = 128
    outs = []
    for p, (s, c) in enumerate(zip(starts_c, parts_c)):
        bo = s // nbp
        cn_p = _conv(orn, orn, dtn, onn, oen_parts[p], idsn_f, time_w, time_b,
                     *p0, nb=nbp, n=c, offs=(bo, bo, bo, bo, 0, bo))
        bq = c // K
        b0 = (s // K) // bq
        cn3_p = jnp.pad(cn_p.reshape(bq, K, NF),
                        ((0, 0), (0, K4 - K), (0, 0))).reshape(-1, NF)
        outs.append(_conv(conv_b, ob, dtb, cn3_p, oeb, idsb_f, time_w, time_b,
                          *p1, nb=bq, n=bq,
                          offs=(b0, b0, b0, 0, b0, b0)))
    return jnp.concatenate(outs, axis=0)
